# baseline (speedup 1.0000x reference)
import functools

import jax
import jax.numpy as jnp
from jax import lax
from jax.experimental import pallas as pl
from jax.experimental.pallas import tpu as pltpu
from jax.experimental.pallas import tpu_sc as plsc

DIM = 64
B = 16384
L2_REG = 1e-4

NC = 2
NS = 16
L = 16
NW = NC * NS
BPW = B // NW
HALF = BPW // 2
HGROUPS = HALF // L


def _sc_scores(u2, i2, gu, gp, gn, pu, pp, pn):
    mesh = plsc.VectorSubcoreMesh(core_axis_name="c", subcore_axis_name="s")

    @functools.partial(
        pl.kernel,
        mesh=mesh,
        compiler_params=pltpu.CompilerParams(needs_layout_passes=False),
        out_type=(
            jax.ShapeDtypeStruct((B,), jnp.float32),
            jax.ShapeDtypeStruct((B,), jnp.float32),
            jax.ShapeDtypeStruct((NW, L), jnp.float32),
        ),
        scratch_types=[
            pltpu.VMEM((BPW,), jnp.int32),
            pltpu.VMEM((BPW,), jnp.int32),
            pltpu.VMEM((BPW,), jnp.int32),
            pltpu.VMEM((BPW,), jnp.int32),
            pltpu.VMEM((BPW,), jnp.int32),
            pltpu.VMEM((BPW,), jnp.int32),
            pltpu.VMEM((HALF, 2 * DIM), jnp.float32),
            pltpu.VMEM((HALF, 2 * DIM), jnp.float32),
            pltpu.VMEM((HALF, 2 * DIM), jnp.float32),
            pltpu.VMEM((BPW,), jnp.float32),
            pltpu.VMEM((BPW,), jnp.float32),
            pltpu.VMEM((L,), jnp.float32),
            pltpu.SemaphoreType.DMA,
            pltpu.SemaphoreType.DMA,
            pltpu.SemaphoreType.DMA,
        ],
    )
    def k(gu_h, gp_h, gn_h, pu_h, pp_h, pn_h, u2_h, i2_h,
          pos_out, neg_out, sq_out,
          guv, gpv, gnv, puv, ppv, pnv, ubuf, pbuf, nbuf, psc, nsc, sqv,
          sem_u, sem_p, sem_n):
        wid = lax.axis_index("s") * NC + lax.axis_index("c")
        base = wid * BPW
        pltpu.sync_copy(gu_h.at[pl.ds(base, BPW)], guv)
        pltpu.sync_copy(gp_h.at[pl.ds(base, BPW)], gpv)
        pltpu.sync_copy(gn_h.at[pl.ds(base, BPW)], gnv)
        pltpu.sync_copy(pu_h.at[pl.ds(base, BPW)], puv)
        pltpu.sync_copy(pp_h.at[pl.ds(base, BPW)], ppv)
        pltpu.sync_copy(pn_h.at[pl.ds(base, BPW)], pnv)

        lanes = lax.iota(jnp.int32, L)

        def half(h, sq):
            hbase = pl.multiple_of(h * HALF, HALF)
            cu = pltpu.async_copy(u2_h.at[guv.at[pl.ds(hbase, HALF)]], ubuf, sem_u)
            cp = pltpu.async_copy(i2_h.at[gpv.at[pl.ds(hbase, HALF)]], pbuf, sem_p)
            cn = pltpu.async_copy(i2_h.at[gnv.at[pl.ds(hbase, HALF)]], nbuf, sem_n)
            cu.wait()
            cp.wait()
            cn.wait()

            def group(g, sq):
                gbase = pl.multiple_of(g * L, L)
                rows = lanes + gbase
                cu_col = puv[pl.ds(hbase + gbase, L)] * DIM
                cp_col = ppv[pl.ds(hbase + gbase, L)] * DIM
                cn_col = pnv[pl.ds(hbase + gbase, L)] * DIM
                pos_v = jnp.zeros((L,), jnp.float32)
                neg_v = jnp.zeros((L,), jnp.float32)
                for j in range(DIM):
                    jv = jnp.full((L,), j, jnp.int32)
                    u = plsc.load_gather(ubuf, [rows, cu_col + jv])
                    p = plsc.load_gather(pbuf, [rows, cp_col + jv])
                    n = plsc.load_gather(nbuf, [rows, cn_col + jv])
                    pos_v = pos_v + u * p
                    neg_v = neg_v + u * n
                    sq = sq + (u * u + (p * p + n * n))
                psc[pl.ds(hbase + gbase, L)] = pos_v
                nsc[pl.ds(hbase + gbase, L)] = neg_v
                return sq

            return lax.fori_loop(0, HGROUPS, group, sq)

        sq = lax.fori_loop(0, 2, half, jnp.zeros((L,), jnp.float32))
        sqv[...] = sq
        pltpu.sync_copy(psc, pos_out.at[pl.ds(base, BPW)])
        pltpu.sync_copy(nsc, neg_out.at[pl.ds(base, BPW)])
        pltpu.sync_copy(sqv, sq_out.at[wid])

    return k(gu, gp, gn, pu, pp, pn, u2, i2)


def _tc_finalize(pos2, neg2, sq2):
    def body(p_ref, n_ref, s_ref, bpr_ref, auc_ref, reg_ref):
        p = p_ref[...]
        n = n_ref[...]
        d = n - p
        sp = jnp.maximum(d, 0.0) + jnp.log(1.0 + jnp.exp(-jnp.abs(d)))
        bpr_ref[0, 0] = jnp.sum(sp) * (1.0 / B)
        auc_ref[0, 0] = jnp.sum((p > n).astype(jnp.float32)) * (1.0 / B)
        reg_ref[0, 0] = (0.5 * L2_REG / B) * jnp.sum(s_ref[...])

    return pl.pallas_call(
        body,
        out_shape=(
            jax.ShapeDtypeStruct((1, 1), jnp.float32),
            jax.ShapeDtypeStruct((1, 1), jnp.float32),
            jax.ShapeDtypeStruct((1, 1), jnp.float32),
        ),
        out_specs=(
            pl.BlockSpec(memory_space=pltpu.SMEM),
            pl.BlockSpec(memory_space=pltpu.SMEM),
            pl.BlockSpec(memory_space=pltpu.SMEM),
        ),
    )(pos2, neg2, sq2)


def kernel(user_table, item_table, users_id, pos_items_id, neg_items_id):
    uid = users_id.astype(jnp.int32)
    pid = pos_items_id.astype(jnp.int32)
    nid = neg_items_id.astype(jnp.int32)
    u2 = user_table.reshape(500000, 2 * DIM)
    i2 = item_table.reshape(500000, 2 * DIM)
    pos_s, neg_s, sq = _sc_scores(
        u2, i2,
        uid >> 1, pid >> 1, nid >> 1,
        uid & 1, pid & 1, nid & 1,
    )
    bpr, auc, reg = _tc_finalize(
        pos_s.reshape(128, 128), neg_s.reshape(128, 128), sq.reshape(4, 128)
    )
    return (bpr[0, 0], auc[0, 0], reg[0, 0])
